# Initial kernel scaffold; baseline (speedup 1.0000x reference)
import jax
import jax.numpy as jnp
from jax.experimental import pallas as pl


def kernel(x, edge_index, batch, W1, b1, W2, b2):
    raise NotImplementedError("write your pallas kernel here")


# speedup vs baseline: 24.6799x; 24.6799x over previous
import functools

import jax
import jax.numpy as jnp
from jax import lax
from jax.experimental import pallas as pl
from jax.experimental.pallas import tpu as pltpu
from jax.experimental.pallas import tpu_sc as plsc

N = 10000
E = 160000
D = 256
H = 512

LANES = 128
NPAD = 10240
ROWS = NPAD // LANES

NC = 2
NS = 16
NW = NC * NS
CH = 40
EPAD = NW * CH * LANES
PER_TILE = NPAD // NS

ROW_BLK = 1000


def _mlp_body(x_ref, w1_ref, b1_ref, w2_ref, s_ref):
    h = jnp.dot(x_ref[...], w1_ref[...], preferred_element_type=jnp.float32)
    h = jnp.maximum(h + b1_ref[...], 0.0)
    s_ref[...] = jnp.sum(h * w2_ref[...], axis=1, keepdims=True)


def _node_scores(x, W1, b1, W2):
    b1r = b1.reshape(1, H)
    w2r = W2.reshape(1, H)
    return pl.pallas_call(
        _mlp_body,
        grid=(N // ROW_BLK,),
        in_specs=[
            pl.BlockSpec((ROW_BLK, D), lambda i: (i, 0)),
            pl.BlockSpec((D, H), lambda i: (0, 0)),
            pl.BlockSpec((1, H), lambda i: (0, 0)),
            pl.BlockSpec((1, H), lambda i: (0, 0)),
        ],
        out_specs=pl.BlockSpec((ROW_BLK, 1), lambda i: (i, 0)),
        out_shape=jax.ShapeDtypeStruct((N, 1), jnp.float32),
    )(x, W1, b1r, w2r)


def _segsum_body(s_hbm, src_hbm, dst_hbm, out_hbm,
                 src_v, dst_v, vals_v, zbuf, agg_sh, sem):
    cid = lax.axis_index("c")
    sid = lax.axis_index("s")
    wid = cid * NS + sid

    for i in range(PER_TILE // 16):
        zbuf[pl.ds(i * 16, 16)] = jnp.zeros((16,), jnp.float32)
    pltpu.sync_copy(zbuf, agg_sh.at[pl.ds(sid * PER_TILE, PER_TILE)])

    pltpu.sync_copy(src_hbm.at[wid], src_v)
    pltpu.sync_copy(dst_hbm.at[wid], dst_v)

    copies = [
        pltpu.make_async_copy(s_hbm.at[src_v.at[j]], vals_v.at[j], sem)
        for j in range(CH)
    ]
    for c in copies:
        c.start()
    for c in copies:
        c.wait()

    plsc.subcore_barrier()

    for j in range(CH):
        pltpu.sync_copy(vals_v.at[j], agg_sh.at[dst_v.at[j]], add=True)

    plsc.subcore_barrier()

    pltpu.sync_copy(agg_sh.at[pl.ds(sid * PER_TILE, PER_TILE)],
                    out_hbm.at[cid, pl.ds(sid * PER_TILE, PER_TILE)])


@functools.cache
def _segsum():
    return pl.kernel(
        _segsum_body,
        out_type=jax.ShapeDtypeStruct((NC, NPAD), jnp.float32),
        mesh=plsc.VectorSubcoreMesh(core_axis_name="c", subcore_axis_name="s",
                                    num_cores=NC, num_subcores=NS),
        scratch_types=[
            pltpu.VMEM((CH, LANES), jnp.int32),
            pltpu.VMEM((CH, LANES), jnp.int32),
            pltpu.VMEM((CH, LANES), jnp.float32),
            pltpu.VMEM((PER_TILE,), jnp.float32),
            pltpu.VMEM_SHARED((NPAD,), jnp.float32),
            pltpu.SemaphoreType.DMA,
        ],
    )


def _finalize_body(s_ref, agg_ref, b2_ref, g_ref, pred_ref, act_ref, lp_ref):
    pred = s_ref[...] + agg_ref[0] + agg_ref[1] + b2_ref[0, 0]
    pred_ref[...] = pred

    row = lax.broadcasted_iota(jnp.int32, (ROWS, LANES), 0)
    col = lax.broadcasted_iota(jnp.int32, (ROWS, LANES), 1)
    flat = row * LANES + col
    valid = flat < N
    neg_inf = jnp.float32(-jnp.inf)

    m = jnp.max(jnp.where(valid, pred, neg_inf))
    e = jnp.where(valid, jnp.exp(pred - m), 0.0)
    p = e / jnp.sum(e)

    y = jnp.where(valid, jnp.log(p + 1e-20) + g_ref[...], neg_inf)
    ymax = jnp.max(y)
    action = jnp.min(jnp.where(y == ymax, flat, jnp.int32(2147483647)))
    act_ref[0, 0] = action
    lp_ref[0, 0] = jnp.log(jnp.sum(jnp.where(flat == action, p, 0.0)))


def _finalize(s80, agg, b2, g80):
    return pl.pallas_call(
        _finalize_body,
        in_specs=[
            pl.BlockSpec(memory_space=pltpu.VMEM),
            pl.BlockSpec(memory_space=pltpu.VMEM),
            pl.BlockSpec(memory_space=pltpu.SMEM),
            pl.BlockSpec(memory_space=pltpu.VMEM),
        ],
        out_specs=[
            pl.BlockSpec(memory_space=pltpu.VMEM),
            pl.BlockSpec(memory_space=pltpu.SMEM),
            pl.BlockSpec(memory_space=pltpu.SMEM),
        ],
        out_shape=[
            jax.ShapeDtypeStruct((ROWS, LANES), jnp.float32),
            jax.ShapeDtypeStruct((1, 1), jnp.int32),
            jax.ShapeDtypeStruct((1, 1), jnp.float32),
        ],
    )(s80, agg, b2.reshape(1, 1), g80)


def kernel(x, edge_index, batch, W1, b1, W2, b2):
    s = _node_scores(x, W1, b1, W2)
    s_pad = jnp.concatenate([s[:, 0], jnp.zeros((NPAD - N,), jnp.float32)])

    src = jnp.concatenate(
        [edge_index[0], jnp.zeros((EPAD - E,), jnp.int32)]).reshape(NW, CH, LANES)
    dst = jnp.concatenate(
        [edge_index[1], jnp.full((EPAD - E,), N, jnp.int32)]).reshape(NW, CH, LANES)

    agg = _segsum()(s_pad, src, dst)

    g = jax.random.gumbel(jax.random.key(42), (N,), jnp.float32)
    g80 = jnp.concatenate([g, jnp.zeros((NPAD - N,), jnp.float32)]
                          ).reshape(ROWS, LANES)

    pred80, act, lp = _finalize(
        s_pad.reshape(ROWS, LANES), agg.reshape(NC, ROWS, LANES), b2, g80)

    pred = pred80.reshape(NPAD, 1)[:N]
    return (pred, act.reshape(()), lp.reshape(()))
